# baseline (speedup 1.0000x reference)
import jax
import jax.numpy as jnp
from jax import lax
from jax.experimental import pallas as pl
from jax.experimental.pallas import tpu as pltpu
from jax.experimental.pallas import tpu_sc as plsc

B, N, D = 4, 2048, 1024
H = D // 4
KEEP = int(N * 0.7)

_ROWS = 512
_GRID = (B * N) // _ROWS


def _scorer_body(x_ref, w1_ref, b1_ref, w2_ref, b2_ref, out_ref):
    xb = x_ref[...]
    h = lax.dot_general(xb, w1_ref[...], (((1,), (1,)), ((), ())),
                        preferred_element_type=jnp.float32)
    h = jnp.maximum(h + b1_ref[...], 0.0)
    hb = h.astype(jnp.bfloat16).astype(jnp.float32)
    wb = w2_ref[...].astype(jnp.bfloat16).astype(jnp.float32)
    z = jnp.sum(hb * wb, axis=1) + b2_ref[0, 0]
    s = jax.nn.sigmoid(z)
    out_ref[...] = s.reshape(out_ref.shape)


def _scores_tc(x2, W1, b1, W2, b2):
    return pl.pallas_call(
        _scorer_body,
        grid=(_GRID,),
        in_specs=[
            pl.BlockSpec((_ROWS, D), lambda i: (i, 0)),
            pl.BlockSpec((H, D), lambda i: (0, 0)),
            pl.BlockSpec((1, H), lambda i: (0, 0)),
            pl.BlockSpec((1, H), lambda i: (0, 0)),
            pl.BlockSpec((1, 1), lambda i: (0, 0)),
        ],
        out_specs=pl.BlockSpec((1, _ROWS // 128, 128), lambda i: (i, 0, 0)),
        out_shape=jax.ShapeDtypeStruct((_GRID, _ROWS // 128, 128),
                                       jnp.float32),
    )(x2, W1, b1.reshape(1, H), W2, b2.reshape(1, 1)).reshape(
        B * N // 128, 128)


def _thresh_body(s_ref, auxf_ref, auxi_ref, loss_ref):
    s = s_ref[...]
    bits = lax.bitcast_convert_type(s, jnp.int32).reshape(B, N // 128, 128)

    def step(k, t):
        cand = t | (1 << (30 - k))
        cnt = jnp.sum((bits >= cand).astype(jnp.int32), axis=(1, 2),
                      keepdims=True)
        return jnp.where(cnt >= KEEP, cand, t)

    t = lax.fori_loop(0, 31, step, jnp.zeros((B, 1, 1), jnp.int32))
    n_gt = jnp.sum((bits > t).astype(jnp.int32), axis=(1, 2), keepdims=True)
    ties = KEEP - n_gt
    thr_f = lax.bitcast_convert_type(t, jnp.float32)
    auxf_ref[...] = jnp.broadcast_to(thr_f.reshape(B, 1), (B, 128))
    auxi_ref[...] = jnp.broadcast_to(ties.reshape(B, 1), (B, 128))
    loss_ref[...] = (-jnp.mean(jnp.abs(s - 0.5))).reshape(1, 1)


def _thresh_tc(scores2d):
    return pl.pallas_call(
        _thresh_body,
        out_shape=(jax.ShapeDtypeStruct((B, 128), jnp.float32),
                   jax.ShapeDtypeStruct((B, 128), jnp.int32),
                   jax.ShapeDtypeStruct((1, 1), jnp.float32)),
    )(scores2d)


_NC, _NS = 2, 16
_TPB = 8
_Q = 192
_G = 16
_PAD = 1600


def _sc_body(x_hbm, scores_hbm, auxf_hbm, auxi_hbm, out_hbm,
             auxf_v, auxi_v, scores_v, idx_full, sc_full,
             shared_idx, shared_sc, idx_l, sc_l, bufs, drain_v,
             gsem, ssem):
    c = lax.axis_index("c")
    s = lax.axis_index("s")
    b_in_core = s // _TPB
    tt = s % _TPB
    batch = 2 * c + b_in_core

    @pl.when(tt == 0)
    def _phase_a():
        pltpu.sync_copy(auxf_hbm, auxf_v)
        pltpu.sync_copy(auxi_hbm, auxi_v)
        pltpu.sync_copy(scores_hbm.at[pl.ds(batch * N, N)], scores_v)
        thr_vec = auxf_v[pl.ds(batch * 128, _NS)]
        ties_vec = auxi_v[pl.ds(batch * 128, _NS)]

        pad_row = jnp.full((_NS,), batch * N, jnp.int32)
        pad_sc = jnp.zeros((_NS,), jnp.float32)
        for off in range(1424, _PAD, _NS):
            idx_full[pl.ds(off, _NS)] = pad_row
            sc_full[pl.ds(off, _NS)] = pad_sc

        def body(i, carry):
            off, nties = carry
            sv = scores_v[pl.ds(i * _NS, _NS)]
            gt = sv > thr_vec
            eq = sv == thr_vec
            eqc = jnp.cumsum(eq.astype(jnp.int32))
            keep = gt | (eq & ((nties + eqc) <= ties_vec))
            kint = keep.astype(jnp.int32)
            pos = off + jnp.cumsum(kint) - 1
            rowid = batch * N + i * _NS + lax.iota(jnp.int32, _NS)
            plsc.store_scatter(idx_full, [pos], rowid, mask=keep)
            plsc.store_scatter(sc_full, [pos], sv, mask=keep)
            return off + jnp.sum(kint), nties + jnp.sum(eq.astype(jnp.int32))

        lax.fori_loop(0, N // _NS, body, (jnp.int32(0), jnp.int32(0)))
        pltpu.sync_copy(idx_full, shared_idx.at[pl.ds(b_in_core * _PAD, _PAD)])
        pltpu.sync_copy(sc_full, shared_sc.at[pl.ds(b_in_core * _PAD, _PAD)])

    plsc.subcore_barrier()

    sh_off = b_in_core * _PAD + tt * _Q
    pltpu.sync_copy(shared_idx.at[pl.ds(sh_off, _Q + _NS)], idx_l)
    pltpu.sync_copy(shared_sc.at[pl.ds(sh_off, _Q + _NS)], sc_l)
    count = jnp.minimum(_Q, KEEP - tt * _Q)
    n_full = count // _G
    rem = count % _G
    out_base = batch * KEEP + tt * _Q

    def start_gather(i, slot):
        pltpu.async_copy(x_hbm.at[idx_l.at[pl.ds(i * _G, _G)]],
                         bufs.at[slot], gsem.at[slot])

    def wait_gather(slot):
        pltpu.make_async_copy(x_hbm.at[pl.ds(0, _G)], bufs.at[slot],
                              gsem.at[slot]).wait()

    def wait_scatter(slot):
        pltpu.make_async_copy(out_hbm.at[pl.ds(0, _G * D)], drain_v,
                              ssem.at[slot]).wait()

    def scale_rows(slot, base):
        def rowbody(j, _):
            sbc = plsc.load_gather(
                sc_l, [jnp.full((_NS,), base + j, jnp.int32)])
            for cc in range(D // _NS):
                sl = pl.ds(cc * _NS, _NS)
                bufs[slot, j, sl] = bufs[slot, j, sl] * sbc
            return 0
        lax.fori_loop(0, _G, rowbody, 0)

    start_gather(0, 0)

    def chunk(i, _):
        slot = i % 2
        other = 1 - slot

        @pl.when(i >= 1)
        def _():
            wait_scatter(other)

        @pl.when(i + 1 < n_full)
        def _():
            start_gather(i + 1, other)

        wait_gather(slot)
        scale_rows(slot, i * _G)
        for j in range(_G):
            pltpu.async_copy(
                bufs.at[slot, j],
                out_hbm.at[pl.ds((out_base + i * _G + j) * D, D)],
                ssem.at[slot])
        return 0

    lax.fori_loop(0, n_full, chunk, 0)
    wait_scatter((n_full - 1) % 2)

    @pl.when(rem > 0)
    def _tail():
        start_gather(n_full, 0)
        wait_gather(0)
        scale_rows(0, n_full * _G)

        def rowcopy(r, _):
            pltpu.sync_copy(
                bufs.at[0, r],
                out_hbm.at[pl.ds((out_base + n_full * _G + r) * D, D)])
            return 0
        lax.fori_loop(0, rem, rowcopy, 0)


def _gather_sc(x2, scores1, auxf1, auxi1):
    return pl.kernel(
        _sc_body,
        out_type=jax.ShapeDtypeStruct((B * KEEP * D,), jnp.float32),
        mesh=plsc.VectorSubcoreMesh(core_axis_name="c", subcore_axis_name="s",
                                    num_cores=_NC, num_subcores=_NS),
        scratch_types=[
            pltpu.VMEM((B * 128,), jnp.float32),
            pltpu.VMEM((B * 128,), jnp.int32),
            pltpu.VMEM((N,), jnp.float32),
            pltpu.VMEM((_PAD,), jnp.int32),
            pltpu.VMEM((_PAD,), jnp.float32),
            pltpu.VMEM_SHARED((2 * _PAD,), jnp.int32),
            pltpu.VMEM_SHARED((2 * _PAD,), jnp.float32),
            pltpu.VMEM((_Q + _NS,), jnp.int32),
            pltpu.VMEM((_Q + _NS,), jnp.float32),
            pltpu.VMEM((2, _G, D), jnp.float32),
            pltpu.VMEM((_G * D,), jnp.float32),
            pltpu.SemaphoreType.DMA((2,)),
            pltpu.SemaphoreType.DMA((2,)),
        ],
        compiler_params=pltpu.CompilerParams(needs_layout_passes=False,
                                             use_tc_tiling_on_sc=True),
    )(x2, scores1, auxf1, auxi1)


def kernel(x, W1, b1, W2, b2):
    x2 = x.reshape(B * N, D)
    scores2d = _scores_tc(x2, W1, b1, W2, b2)
    auxf, auxi, loss = _thresh_tc(scores2d)
    x_kept = _gather_sc(x2, scores2d.reshape(B * N),
                        auxf.reshape(B * 128), auxi.reshape(B * 128))
    return (x_kept.reshape(B, KEEP, D), loss.reshape(()))


# speedup vs baseline: 1.6428x; 1.6428x over previous
import jax
import jax.numpy as jnp
from jax import lax
from jax.experimental import pallas as pl
from jax.experimental.pallas import tpu as pltpu
from jax.experimental.pallas import tpu_sc as plsc

B, N, D = 4, 2048, 1024
H = D // 4
KEEP = int(N * 0.7)

_ROWS = 512
_GRID = (B * N) // _ROWS


def _scorer_body(x_ref, w1_ref, b1_ref, w2_ref, b2_ref, out_ref):
    xb = x_ref[...]
    h = lax.dot_general(xb, w1_ref[...], (((1,), (1,)), ((), ())),
                        preferred_element_type=jnp.float32)
    h = jnp.maximum(h + b1_ref[...], 0.0)
    hb = h.astype(jnp.bfloat16).astype(jnp.float32)
    wb = w2_ref[...].astype(jnp.bfloat16).astype(jnp.float32)
    z = jnp.sum(hb * wb, axis=1) + b2_ref[0, 0]
    s = jax.nn.sigmoid(z)
    out_ref[...] = s.reshape(out_ref.shape)


def _scores_tc(x2, W1, b1, W2, b2):
    return pl.pallas_call(
        _scorer_body,
        grid=(_GRID,),
        in_specs=[
            pl.BlockSpec((_ROWS, D), lambda i: (i, 0)),
            pl.BlockSpec((H, D), lambda i: (0, 0)),
            pl.BlockSpec((1, H), lambda i: (0, 0)),
            pl.BlockSpec((1, H), lambda i: (0, 0)),
            pl.BlockSpec((1, 1), lambda i: (0, 0)),
        ],
        out_specs=pl.BlockSpec((1, _ROWS // 128, 128), lambda i: (i, 0, 0)),
        out_shape=jax.ShapeDtypeStruct((_GRID, _ROWS // 128, 128),
                                       jnp.float32),
    )(x2, W1, b1.reshape(1, H), W2, b2.reshape(1, 1)).reshape(
        B * N // 128, 128)


def _thresh_body(s_ref, auxf_ref, auxi_ref, loss_ref):
    s = s_ref[...]
    bits = lax.bitcast_convert_type(s, jnp.int32).reshape(B, N // 128, 128)

    def step(k, t):
        cand = t | (1 << (30 - k))
        cnt = jnp.sum((bits >= cand).astype(jnp.int32), axis=(1, 2),
                      keepdims=True)
        return jnp.where(cnt >= KEEP, cand, t)

    t = lax.fori_loop(0, 31, step, jnp.zeros((B, 1, 1), jnp.int32))
    n_gt = jnp.sum((bits > t).astype(jnp.int32), axis=(1, 2), keepdims=True)
    ties = KEEP - n_gt
    thr_f = lax.bitcast_convert_type(t, jnp.float32)
    auxf_ref[...] = jnp.broadcast_to(thr_f.reshape(B, 1), (B, 128))
    auxi_ref[...] = jnp.broadcast_to(ties.reshape(B, 1), (B, 128))
    loss_ref[...] = (-jnp.mean(jnp.abs(s - 0.5))).reshape(1, 1)


def _thresh_tc(scores2d):
    return pl.pallas_call(
        _thresh_body,
        out_shape=(jax.ShapeDtypeStruct((B, 128), jnp.float32),
                   jax.ShapeDtypeStruct((B, 128), jnp.int32),
                   jax.ShapeDtypeStruct((1, 1), jnp.float32)),
    )(scores2d)


_NC, _NS = 2, 16
_TPB = 8
_Q = 192
_G = 16
_PAD = 1600


def _sc_body(x_hbm, scores_hbm, auxf_hbm, auxi_hbm, out_hbm,
             auxf_v, auxi_v, scores_v, idx_full, sc_full,
             shared_idx, shared_sc, idx_l, sc_l, bufs, drain_v, gidx,
             gsem, ssem):
    c = lax.axis_index("c")
    s = lax.axis_index("s")
    b_in_core = s // _TPB
    tt = s % _TPB
    batch = 2 * c + b_in_core

    @pl.when(tt == 0)
    def _phase_a():
        pltpu.sync_copy(auxf_hbm, auxf_v)
        pltpu.sync_copy(auxi_hbm, auxi_v)
        pltpu.sync_copy(scores_hbm.at[pl.ds(batch * N, N)], scores_v)
        thr_vec = auxf_v[pl.ds(batch * 128, _NS)]
        ties_vec = auxi_v[pl.ds(batch * 128, _NS)]

        pad_row = jnp.full((_NS,), batch * N, jnp.int32)
        pad_sc = jnp.zeros((_NS,), jnp.float32)
        for off in range(1424, _PAD, _NS):
            idx_full[pl.ds(off, _NS)] = pad_row
            sc_full[pl.ds(off, _NS)] = pad_sc

        def body(i, carry):
            off, nties = carry
            sv = scores_v[pl.ds(i * _NS, _NS)]
            gt = sv > thr_vec
            eq = sv == thr_vec
            eqc = jnp.cumsum(eq.astype(jnp.int32))
            keep = gt | (eq & ((nties + eqc) <= ties_vec))
            kint = keep.astype(jnp.int32)
            pos = off + jnp.cumsum(kint) - 1
            rowid = batch * N + i * _NS + lax.iota(jnp.int32, _NS)
            plsc.store_scatter(idx_full, [pos], rowid, mask=keep)
            plsc.store_scatter(sc_full, [pos], sv, mask=keep)
            return off + jnp.sum(kint), nties + jnp.sum(eq.astype(jnp.int32))

        lax.fori_loop(0, N // _NS, body, (jnp.int32(0), jnp.int32(0)))
        pltpu.sync_copy(idx_full, shared_idx.at[pl.ds(b_in_core * _PAD, _PAD)])
        pltpu.sync_copy(sc_full, shared_sc.at[pl.ds(b_in_core * _PAD, _PAD)])

    plsc.subcore_barrier()

    sh_off = b_in_core * _PAD + tt * _Q
    pltpu.sync_copy(shared_idx.at[pl.ds(sh_off, _Q + _NS)], idx_l)
    pltpu.sync_copy(shared_sc.at[pl.ds(sh_off, _Q + _NS)], sc_l)
    count = jnp.minimum(_Q, KEEP - tt * _Q)
    n_full = count // _G
    rem = count % _G
    out_base = tt * _Q

    def start_gather(i, slot):
        idx16 = idx_l[pl.ds(i * _G, _G)]
        base = (idx16 >> 3) * 64 + (idx16 & 7)
        for c in range(D // 128):
            gidx[slot, pl.ds(c * _G, _G)] = base + c * 8
        pltpu.async_copy(x_hbm.at[gidx.at[slot]], bufs.at[slot],
                         gsem.at[slot])

    def wait_gather(slot):
        pltpu.make_async_copy(x_hbm.at[pl.ds(0, 8 * _G)], bufs.at[slot],
                              gsem.at[slot]).wait()

    def wait_scatter(slot):
        pltpu.make_async_copy(x_hbm.at[pl.ds(0, 8 * _G)], drain_v,
                              ssem.at[slot]).wait()

    def scale_rows(slot, base):
        def rowbody(j, _):
            sbc = plsc.load_gather(
                sc_l, [jnp.full((_NS,), base + j, jnp.int32)])
            for c in range(D // 128):
                for k2 in range(128 // _NS):
                    sl = pl.ds(k2 * _NS, _NS)
                    r = c * _G + j
                    bufs[slot, r, sl] = bufs[slot, r, sl] * sbc
            return 0
        lax.fori_loop(0, _G, rowbody, 0)

    start_gather(0, 0)

    def chunk(i, _):
        slot = i % 2
        other = 1 - slot

        @pl.when(i >= 1)
        def _():
            wait_scatter(other)

        @pl.when(i + 1 < n_full)
        def _():
            start_gather(i + 1, other)

        wait_gather(slot)
        scale_rows(slot, i * _G)
        for c in range(D // 128):
            pltpu.async_copy(
                bufs.at[slot, pl.ds(c * _G, _G)],
                out_hbm.at[pl.ds(out_base + i * _G, _G), c, batch],
                ssem.at[slot])
        return 0

    lax.fori_loop(0, n_full, chunk, 0)
    wait_scatter((n_full - 1) % 2)

    @pl.when(rem > 0)
    def _tail():
        start_gather(n_full, 0)
        wait_gather(0)
        scale_rows(0, n_full * _G)

        def rowcopy(r, _):
            for c in range(D // 128):
                pltpu.sync_copy(
                    bufs.at[0, c * _G + r],
                    out_hbm.at[out_base + n_full * _G + r, c, batch])
            return 0
        lax.fori_loop(0, rem, rowcopy, 0)


def _gather_sc(x2, scores1, auxf1, auxi1):
    return pl.kernel(
        _sc_body,
        out_type=jax.ShapeDtypeStruct((KEEP, D // 128, B, 128),
                                      jnp.float32),
        mesh=plsc.VectorSubcoreMesh(core_axis_name="c", subcore_axis_name="s",
                                    num_cores=_NC, num_subcores=_NS),
        scratch_types=[
            pltpu.VMEM((B * 128,), jnp.float32),
            pltpu.VMEM((B * 128,), jnp.int32),
            pltpu.VMEM((N,), jnp.float32),
            pltpu.VMEM((_PAD,), jnp.int32),
            pltpu.VMEM((_PAD,), jnp.float32),
            pltpu.VMEM_SHARED((2 * _PAD,), jnp.int32),
            pltpu.VMEM_SHARED((2 * _PAD,), jnp.float32),
            pltpu.VMEM((_Q + _NS,), jnp.int32),
            pltpu.VMEM((_Q + _NS,), jnp.float32),
            pltpu.VMEM((2, 8 * _G, 128), jnp.float32),
            pltpu.VMEM((8 * _G, 128), jnp.float32),
            pltpu.VMEM((2, 8 * _G), jnp.int32),
            pltpu.SemaphoreType.DMA((2,)),
            pltpu.SemaphoreType.DMA((2,)),
        ],
        compiler_params=pltpu.CompilerParams(needs_layout_passes=False,
                                             use_tc_tiling_on_sc=False),
    )(x2, scores1, auxf1, auxi1)


def kernel(x, W1, b1, W2, b2):
    x2 = x.reshape(B * N, D)
    xg = x.reshape(B * N // 8, 8, D // 128, 128).transpose(0, 2, 1, 3)
    xg = xg.reshape(B * N * 8, 128)
    scores2d = _scores_tc(x2, W1, b1, W2, b2)
    auxf, auxi, loss = _thresh_tc(scores2d)
    x_kept = _gather_sc(xg, scores2d.reshape(B * N),
                        auxf.reshape(B * 128), auxi.reshape(B * 128))
    x_kept = x_kept.transpose(2, 0, 1, 3).reshape(B, KEEP, D)
    return (x_kept, loss.reshape(()))


# speedup vs baseline: 1.7733x; 1.0795x over previous
import jax
import jax.numpy as jnp
from jax import lax
from jax.experimental import pallas as pl
from jax.experimental.pallas import tpu as pltpu
from jax.experimental.pallas import tpu_sc as plsc

B, N, D = 4, 2048, 1024
H = D // 4
KEEP = int(N * 0.7)

_ROWS = 1024
_GRID = (B * N) // _ROWS


def _scorer_body(x_ref, w1_ref, b1_ref, w2_ref, b2_ref, out_ref):
    xb = x_ref[...]
    h = lax.dot_general(xb, w1_ref[...], (((1,), (1,)), ((), ())),
                        preferred_element_type=jnp.float32)
    h = jnp.maximum(h + b1_ref[...], 0.0)
    hb = h.astype(jnp.bfloat16).astype(jnp.float32)
    wb = w2_ref[...].astype(jnp.bfloat16).astype(jnp.float32)
    z = jnp.sum(hb * wb, axis=1) + b2_ref[0, 0]
    s = jax.nn.sigmoid(z)
    out_ref[...] = s.reshape(out_ref.shape)


def _scores_tc(x2, W1, b1, W2, b2):
    return pl.pallas_call(
        _scorer_body,
        grid=(_GRID,),
        in_specs=[
            pl.BlockSpec((_ROWS, D), lambda i: (i, 0)),
            pl.BlockSpec((H, D), lambda i: (0, 0)),
            pl.BlockSpec((1, H), lambda i: (0, 0)),
            pl.BlockSpec((1, H), lambda i: (0, 0)),
            pl.BlockSpec((1, 1), lambda i: (0, 0)),
        ],
        out_specs=pl.BlockSpec((1, _ROWS // 128, 128), lambda i: (i, 0, 0)),
        out_shape=jax.ShapeDtypeStruct((_GRID, _ROWS // 128, 128),
                                       jnp.float32),
    )(x2, W1, b1.reshape(1, H), W2, b2.reshape(1, 1)).reshape(
        B * N // 128, 128)


def _thresh_body(s_ref, auxf_ref, auxi_ref, loss_ref):
    s = s_ref[...]
    bits = lax.bitcast_convert_type(s, jnp.int32).reshape(B, N // 128, 128)

    def step(k, t):
        cand = t | (1 << (30 - k))
        cnt = jnp.sum((bits >= cand).astype(jnp.int32), axis=(1, 2),
                      keepdims=True)
        return jnp.where(cnt >= KEEP, cand, t)

    t = lax.fori_loop(0, 31, step, jnp.zeros((B, 1, 1), jnp.int32))
    n_gt = jnp.sum((bits > t).astype(jnp.int32), axis=(1, 2), keepdims=True)
    ties = KEEP - n_gt
    thr_f = lax.bitcast_convert_type(t, jnp.float32)
    auxf_ref[...] = jnp.broadcast_to(thr_f.reshape(B, 1), (B, 128))
    auxi_ref[...] = jnp.broadcast_to(ties.reshape(B, 1), (B, 128))
    loss_ref[...] = (-jnp.mean(jnp.abs(s - 0.5))).reshape(1, 1)


def _thresh_tc(scores2d):
    return pl.pallas_call(
        _thresh_body,
        out_shape=(jax.ShapeDtypeStruct((B, 128), jnp.float32),
                   jax.ShapeDtypeStruct((B, 128), jnp.int32),
                   jax.ShapeDtypeStruct((1, 1), jnp.float32)),
    )(scores2d)


_NC, _NS = 2, 16
_TPB = 8
_Q = 192
_G = 16
_PAD = 1600


def _sc_body(x_hbm, scores_hbm, auxf_hbm, auxi_hbm, out_hbm,
             auxf_v, auxi_v, scores_v, idx_full, sc_full,
             shared_idx, shared_sc, idx_l, sc_l, bufs, drain_v, gidx,
             gsem, ssem):
    c = lax.axis_index("c")
    s = lax.axis_index("s")
    b_in_core = s // _TPB
    tt = s % _TPB
    batch = 2 * c + b_in_core

    @pl.when(tt == 0)
    def _phase_a():
        pltpu.sync_copy(auxf_hbm, auxf_v)
        pltpu.sync_copy(auxi_hbm, auxi_v)
        pltpu.sync_copy(scores_hbm.at[pl.ds(batch * N, N)], scores_v)
        thr_vec = auxf_v[pl.ds(batch * 128, _NS)]
        ties_vec = auxi_v[pl.ds(batch * 128, _NS)]

        pad_row = jnp.full((_NS,), batch * N, jnp.int32)
        pad_sc = jnp.zeros((_NS,), jnp.float32)
        for off in range(1424, _PAD, _NS):
            idx_full[pl.ds(off, _NS)] = pad_row
            sc_full[pl.ds(off, _NS)] = pad_sc

        def body(i, carry):
            off, nties = carry
            sv = scores_v[pl.ds(i * _NS, _NS)]
            gt = sv > thr_vec
            eq = sv == thr_vec
            eqc = jnp.cumsum(eq.astype(jnp.int32))
            keep = gt | (eq & ((nties + eqc) <= ties_vec))
            kint = keep.astype(jnp.int32)
            pos = off + jnp.cumsum(kint) - 1
            rowid = batch * N + i * _NS + lax.iota(jnp.int32, _NS)
            plsc.store_scatter(idx_full, [pos], rowid, mask=keep)
            plsc.store_scatter(sc_full, [pos], sv, mask=keep)
            return off + jnp.sum(kint), nties + jnp.sum(eq.astype(jnp.int32))

        lax.fori_loop(0, N // _NS, body, (jnp.int32(0), jnp.int32(0)))
        pltpu.sync_copy(idx_full, shared_idx.at[pl.ds(b_in_core * _PAD, _PAD)])
        pltpu.sync_copy(sc_full, shared_sc.at[pl.ds(b_in_core * _PAD, _PAD)])

    plsc.subcore_barrier()

    sh_off = b_in_core * _PAD + tt * _Q
    pltpu.sync_copy(shared_idx.at[pl.ds(sh_off, _Q + _NS)], idx_l)
    pltpu.sync_copy(shared_sc.at[pl.ds(sh_off, _Q + _NS)], sc_l)
    count = jnp.minimum(_Q, KEEP - tt * _Q)
    n_full = count // _G
    rem = count % _G
    out_base = tt * _Q

    def start_gather(i, slot):
        idx16 = idx_l[pl.ds(i * _G, _G)]
        base = (idx16 >> 3) * 64 + (idx16 & 7)
        for c in range(D // 128):
            gidx[slot, pl.ds(c * _G, _G)] = base + c * 8
        pltpu.async_copy(x_hbm.at[gidx.at[slot]], bufs.at[slot],
                         gsem.at[slot])

    def wait_gather(slot):
        pltpu.make_async_copy(x_hbm.at[pl.ds(0, 8 * _G)], bufs.at[slot],
                              gsem.at[slot]).wait()

    def wait_scatter(slot):
        pltpu.make_async_copy(x_hbm.at[pl.ds(0, 8 * _G)], drain_v,
                              ssem.at[slot]).wait()

    def scale_rows(slot, base):
        def rowbody(j, _):
            sbc = plsc.load_gather(
                sc_l, [jnp.full((_NS,), base + j, jnp.int32)])
            for c in range(D // 128):
                for k2 in range(128 // _NS):
                    sl = pl.ds(k2 * _NS, _NS)
                    r = c * _G + j
                    bufs[slot, r, sl] = bufs[slot, r, sl] * sbc
            return 0
        lax.fori_loop(0, _G, rowbody, 0)

    start_gather(0, 0)

    def chunk(i, _):
        slot = i % 2
        other = 1 - slot

        @pl.when(i >= 1)
        def _():
            wait_scatter(other)

        @pl.when(i + 1 < n_full)
        def _():
            start_gather(i + 1, other)

        wait_gather(slot)
        scale_rows(slot, i * _G)
        for c in range(D // 128):
            pltpu.async_copy(
                bufs.at[slot, pl.ds(c * _G, _G)],
                out_hbm.at[pl.ds(out_base + i * _G, _G), c, batch],
                ssem.at[slot])
        return 0

    lax.fori_loop(0, n_full, chunk, 0)
    wait_scatter((n_full - 1) % 2)

    @pl.when(rem > 0)
    def _tail():
        start_gather(n_full, 0)
        wait_gather(0)
        scale_rows(0, n_full * _G)

        def rowcopy(r, _):
            for c in range(D // 128):
                pltpu.sync_copy(
                    bufs.at[0, c * _G + r],
                    out_hbm.at[out_base + n_full * _G + r, c, batch])
            return 0
        lax.fori_loop(0, rem, rowcopy, 0)


def _gather_sc(x2, scores1, auxf1, auxi1):
    return pl.kernel(
        _sc_body,
        out_type=jax.ShapeDtypeStruct((KEEP, D // 128, B, 128),
                                      jnp.float32),
        mesh=plsc.VectorSubcoreMesh(core_axis_name="c", subcore_axis_name="s",
                                    num_cores=_NC, num_subcores=_NS),
        scratch_types=[
            pltpu.VMEM((B * 128,), jnp.float32),
            pltpu.VMEM((B * 128,), jnp.int32),
            pltpu.VMEM((N,), jnp.float32),
            pltpu.VMEM((_PAD,), jnp.int32),
            pltpu.VMEM((_PAD,), jnp.float32),
            pltpu.VMEM_SHARED((2 * _PAD,), jnp.int32),
            pltpu.VMEM_SHARED((2 * _PAD,), jnp.float32),
            pltpu.VMEM((_Q + _NS,), jnp.int32),
            pltpu.VMEM((_Q + _NS,), jnp.float32),
            pltpu.VMEM((2, 8 * _G, 128), jnp.float32),
            pltpu.VMEM((8 * _G, 128), jnp.float32),
            pltpu.VMEM((2, 8 * _G), jnp.int32),
            pltpu.SemaphoreType.DMA((2,)),
            pltpu.SemaphoreType.DMA((2,)),
        ],
        compiler_params=pltpu.CompilerParams(needs_layout_passes=False,
                                             use_tc_tiling_on_sc=False),
    )(x2, scores1, auxf1, auxi1)


def kernel(x, W1, b1, W2, b2):
    x2 = x.reshape(B * N, D)
    xg = x.reshape(B * N // 8, 8, D // 128, 128).transpose(0, 2, 1, 3)
    xg = xg.reshape(B * N * 8, 128)
    scores2d = _scores_tc(x2, W1, b1, W2, b2)
    auxf, auxi, loss = _thresh_tc(scores2d)
    x_kept = _gather_sc(xg, scores2d.reshape(B * N),
                        auxf.reshape(B * 128), auxi.reshape(B * 128))
    x_kept = x_kept.transpose(2, 0, 1, 3).reshape(B, KEEP, D)
    return (x_kept, loss.reshape(()))


# speedup vs baseline: 1.8161x; 1.0241x over previous
import jax
import jax.numpy as jnp
from jax import lax
from jax.experimental import pallas as pl
from jax.experimental.pallas import tpu as pltpu
from jax.experimental.pallas import tpu_sc as plsc

B, N, D = 4, 2048, 1024
H = D // 4
KEEP = int(N * 0.7)

_ROWS = 2048
_GRID = (B * N) // _ROWS


def _scorer_body(x_ref, w1_ref, b1_ref, w2_ref, b2_ref, out_ref):
    xb = x_ref[...]
    h = lax.dot_general(xb, w1_ref[...], (((1,), (1,)), ((), ())),
                        preferred_element_type=jnp.float32)
    h = jnp.maximum(h + b1_ref[...], 0.0)
    hb = h.astype(jnp.bfloat16).astype(jnp.float32)
    wb = w2_ref[...].astype(jnp.bfloat16).astype(jnp.float32)
    z = jnp.sum(hb * wb, axis=1) + b2_ref[0, 0]
    s = jax.nn.sigmoid(z)
    out_ref[...] = s.reshape(out_ref.shape)


def _scores_tc(x2, W1, b1, W2, b2):
    return pl.pallas_call(
        _scorer_body,
        grid=(_GRID,),
        in_specs=[
            pl.BlockSpec((_ROWS, D), lambda i: (i, 0)),
            pl.BlockSpec((H, D), lambda i: (0, 0)),
            pl.BlockSpec((1, H), lambda i: (0, 0)),
            pl.BlockSpec((1, H), lambda i: (0, 0)),
            pl.BlockSpec((1, 1), lambda i: (0, 0)),
        ],
        out_specs=pl.BlockSpec((1, _ROWS // 128, 128), lambda i: (i, 0, 0)),
        out_shape=jax.ShapeDtypeStruct((_GRID, _ROWS // 128, 128),
                                       jnp.float32),
    )(x2, W1, b1.reshape(1, H), W2, b2.reshape(1, 1)).reshape(
        B * N // 128, 128)


def _thresh_body(s_ref, auxf_ref, auxi_ref, loss_ref):
    s = s_ref[...]
    bits = lax.bitcast_convert_type(s, jnp.int32).reshape(B, N // 128, 128)

    def step(k, t):
        cand = t | (1 << (30 - k))
        cnt = jnp.sum((bits >= cand).astype(jnp.int32), axis=(1, 2),
                      keepdims=True)
        return jnp.where(cnt >= KEEP, cand, t)

    t = lax.fori_loop(0, 31, step, jnp.zeros((B, 1, 1), jnp.int32))
    n_gt = jnp.sum((bits > t).astype(jnp.int32), axis=(1, 2), keepdims=True)
    ties = KEEP - n_gt
    thr_f = lax.bitcast_convert_type(t, jnp.float32)
    auxf_ref[...] = jnp.broadcast_to(thr_f.reshape(B, 1), (B, 128))
    auxi_ref[...] = jnp.broadcast_to(ties.reshape(B, 1), (B, 128))
    loss_ref[...] = (-jnp.mean(jnp.abs(s - 0.5))).reshape(1, 1)


def _thresh_tc(scores2d):
    return pl.pallas_call(
        _thresh_body,
        out_shape=(jax.ShapeDtypeStruct((B, 128), jnp.float32),
                   jax.ShapeDtypeStruct((B, 128), jnp.int32),
                   jax.ShapeDtypeStruct((1, 1), jnp.float32)),
    )(scores2d)


_NC, _NS = 2, 16
_TPB = 8
_Q = 192
_G = 16
_PAD = 1600


def _sc_body(x_hbm, scores_hbm, auxf_hbm, auxi_hbm, out_hbm,
             auxf_v, auxi_v, scores_v, idx_full, sc_full,
             shared_idx, shared_sc, idx_l, sc_l, bufs, drain_v, gidx,
             gsem, ssem):
    c = lax.axis_index("c")
    s = lax.axis_index("s")
    b_in_core = s // _TPB
    tt = s % _TPB
    batch = 2 * c + b_in_core

    @pl.when(tt == 0)
    def _phase_a():
        pltpu.sync_copy(auxf_hbm, auxf_v)
        pltpu.sync_copy(auxi_hbm, auxi_v)
        pltpu.sync_copy(scores_hbm.at[pl.ds(batch * N, N)], scores_v)
        thr_vec = auxf_v[pl.ds(batch * 128, _NS)]
        ties_vec = auxi_v[pl.ds(batch * 128, _NS)]

        pad_row = jnp.full((_NS,), batch * N, jnp.int32)
        pad_sc = jnp.zeros((_NS,), jnp.float32)
        for off in range(1424, _PAD, _NS):
            idx_full[pl.ds(off, _NS)] = pad_row
            sc_full[pl.ds(off, _NS)] = pad_sc

        def body(i, carry):
            off, nties = carry
            sv = scores_v[pl.ds(i * _NS, _NS)]
            gt = sv > thr_vec
            eq = sv == thr_vec
            eqc = jnp.cumsum(eq.astype(jnp.int32))
            keep = gt | (eq & ((nties + eqc) <= ties_vec))
            kint = keep.astype(jnp.int32)
            pos = off + jnp.cumsum(kint) - 1
            rowid = batch * N + i * _NS + lax.iota(jnp.int32, _NS)
            plsc.store_scatter(idx_full, [pos], rowid, mask=keep)
            plsc.store_scatter(sc_full, [pos], sv, mask=keep)
            return off + jnp.sum(kint), nties + jnp.sum(eq.astype(jnp.int32))

        lax.fori_loop(0, N // _NS, body, (jnp.int32(0), jnp.int32(0)))
        pltpu.sync_copy(idx_full, shared_idx.at[pl.ds(b_in_core * _PAD, _PAD)])
        pltpu.sync_copy(sc_full, shared_sc.at[pl.ds(b_in_core * _PAD, _PAD)])

    plsc.subcore_barrier()

    sh_off = b_in_core * _PAD + tt * _Q
    pltpu.sync_copy(shared_idx.at[pl.ds(sh_off, _Q + _NS)], idx_l)
    pltpu.sync_copy(shared_sc.at[pl.ds(sh_off, _Q + _NS)], sc_l)
    count = jnp.minimum(_Q, KEEP - tt * _Q)
    n_full = count // _G
    rem = count % _G
    out_base = tt * _Q

    def start_gather(i, slot):
        idx16 = idx_l[pl.ds(i * _G, _G)]
        base = (idx16 >> 3) * 64 + (idx16 & 7)
        for c in range(D // 128):
            gidx[slot, pl.ds(c * _G, _G)] = base + c * 8
        pltpu.async_copy(x_hbm.at[gidx.at[slot]], bufs.at[slot],
                         gsem.at[slot])

    def wait_gather(slot):
        pltpu.make_async_copy(x_hbm.at[pl.ds(0, 8 * _G)], bufs.at[slot],
                              gsem.at[slot]).wait()

    def wait_scatter(slot):
        pltpu.make_async_copy(x_hbm.at[pl.ds(0, 8 * _G)], drain_v,
                              ssem.at[slot]).wait()

    def scale_rows(slot, base):
        def rowbody(j, _):
            sbc = plsc.load_gather(
                sc_l, [jnp.full((_NS,), base + j, jnp.int32)])
            for c in range(D // 128):
                for k2 in range(128 // _NS):
                    sl = pl.ds(k2 * _NS, _NS)
                    r = c * _G + j
                    bufs[slot, r, sl] = bufs[slot, r, sl] * sbc
            return 0
        lax.fori_loop(0, _G, rowbody, 0)

    start_gather(0, 0)

    def chunk(i, _):
        slot = i % 2
        other = 1 - slot

        @pl.when(i >= 1)
        def _():
            wait_scatter(other)

        @pl.when(i + 1 < n_full)
        def _():
            start_gather(i + 1, other)

        wait_gather(slot)
        scale_rows(slot, i * _G)
        for c in range(D // 128):
            pltpu.async_copy(
                bufs.at[slot, pl.ds(c * _G, _G)],
                out_hbm.at[pl.ds(out_base + i * _G, _G), c, batch],
                ssem.at[slot])
        return 0

    lax.fori_loop(0, n_full, chunk, 0)
    wait_scatter((n_full - 1) % 2)

    @pl.when(rem > 0)
    def _tail():
        start_gather(n_full, 0)
        wait_gather(0)
        scale_rows(0, n_full * _G)

        def rowcopy(r, _):
            for c in range(D // 128):
                pltpu.sync_copy(
                    bufs.at[0, c * _G + r],
                    out_hbm.at[out_base + n_full * _G + r, c, batch])
            return 0
        lax.fori_loop(0, rem, rowcopy, 0)


def _gather_sc(x2, scores1, auxf1, auxi1):
    return pl.kernel(
        _sc_body,
        out_type=jax.ShapeDtypeStruct((KEEP, D // 128, B, 128),
                                      jnp.float32),
        mesh=plsc.VectorSubcoreMesh(core_axis_name="c", subcore_axis_name="s",
                                    num_cores=_NC, num_subcores=_NS),
        scratch_types=[
            pltpu.VMEM((B * 128,), jnp.float32),
            pltpu.VMEM((B * 128,), jnp.int32),
            pltpu.VMEM((N,), jnp.float32),
            pltpu.VMEM((_PAD,), jnp.int32),
            pltpu.VMEM((_PAD,), jnp.float32),
            pltpu.VMEM_SHARED((2 * _PAD,), jnp.int32),
            pltpu.VMEM_SHARED((2 * _PAD,), jnp.float32),
            pltpu.VMEM((_Q + _NS,), jnp.int32),
            pltpu.VMEM((_Q + _NS,), jnp.float32),
            pltpu.VMEM((2, 8 * _G, 128), jnp.float32),
            pltpu.VMEM((8 * _G, 128), jnp.float32),
            pltpu.VMEM((2, 8 * _G), jnp.int32),
            pltpu.SemaphoreType.DMA((2,)),
            pltpu.SemaphoreType.DMA((2,)),
        ],
        compiler_params=pltpu.CompilerParams(needs_layout_passes=False,
                                             use_tc_tiling_on_sc=False),
    )(x2, scores1, auxf1, auxi1)


def kernel(x, W1, b1, W2, b2):
    x2 = x.reshape(B * N, D)
    xg = x.reshape(B * N // 8, 8, D // 128, 128).transpose(0, 2, 1, 3)
    xg = xg.reshape(B * N * 8, 128)
    scores2d = _scores_tc(x2, W1, b1, W2, b2)
    auxf, auxi, loss = _thresh_tc(scores2d)
    x_kept = _gather_sc(xg, scores2d.reshape(B * N),
                        auxf.reshape(B * 128), auxi.reshape(B * 128))
    x_kept = x_kept.transpose(2, 0, 1, 3).reshape(B, KEEP, D)
    return (x_kept, loss.reshape(()))
